# Initial kernel scaffold; baseline (speedup 1.0000x reference)
import jax
import jax.numpy as jnp
from jax.experimental import pallas as pl


def kernel(x_user, x_item, edge_index_ui, edge_index_iu, edge_attr_ui, edge_attr_iu, params):
    raise NotImplementedError("write your pallas kernel here")


# speedup vs baseline: 12.0043x; 12.0043x over previous
import functools

import jax
import jax.numpy as jnp
from jax import lax
from jax.experimental import pallas as pl
from jax.experimental.pallas import tpu as pltpu
from jax.experimental.pallas import tpu_sc as plsc

H = 128
HEADS = 8
DH = 16
N = 10000
E = 320000

NC = 2
NS = 16
NW = NC * NS
EPW = E // NW
C = 80
NCHUNK = EPW // C
ZROWS = 80
NZCH = N // ZROWS

_f32 = jnp.float32
_i32 = jnp.int32


def _passA_body(q_hbm, k_hbm, qwb_hbm, src_hbm, dst_hbm, ea_hbm,
                acc_out, ex_out,
                srcv, dstv, eav, qrows, krows, qwbrows, exrow, exhm, zbuf,
                acc_sh, sem1, sem2, sem3):
    cid = lax.axis_index("c")
    sid = lax.axis_index("s")
    wid = sid * NC + cid

    def _z(i, _):
        zbuf[i, :] = jnp.zeros((16,), _f32)
        return _
    lax.fori_loop(0, ZROWS, _z, None)
    nzt = (NZCH - sid + NS - 1) // NS

    def _zc(t, _):
        j = sid + NS * t
        pltpu.sync_copy(zbuf, acc_sh.at[pl.ds(j * ZROWS, ZROWS)])
        return _
    lax.fori_loop(0, nzt, _zc, None)
    plsc.subcore_barrier()

    def _chunk(g, _):
        base = wid * EPW + g * C
        pltpu.sync_copy(src_hbm.at[pl.ds(base, C)], srcv)
        pltpu.sync_copy(dst_hbm.at[pl.ds(base, C)], dstv)
        pltpu.sync_copy(ea_hbm.at[pl.ds(base, C)], eav)
        cp1 = pltpu.async_copy(q_hbm.at[dstv], qrows, sem1)
        cp2 = pltpu.async_copy(k_hbm.at[srcv], krows, sem2)
        cp3 = pltpu.async_copy(qwb_hbm.at[dstv], qwbrows, sem3)
        cp1.wait()
        cp2.wait()
        cp3.wait()

        def _grp(t, _):
            eids = lax.iota(_i32, 16) + t * 16
            av = eav[pl.ds(t * 16, 16)]
            for h in range(HEADS):
                hq = jnp.full((16,), h, _i32)
                qwv = plsc.load_gather(qwbrows, [eids, hq])
                qbv = plsc.load_gather(qwbrows, [eids, hq + 8])
                s = qbv + av * qwv
                for d in range(DH):
                    cc = jnp.full((16,), h * DH + d, _i32)
                    qc = plsc.load_gather(qrows, [eids, cc])
                    kc = plsc.load_gather(krows, [eids, cc])
                    s = s + qc * kc
                ex = jnp.exp(s * 0.25)
                exhm[h, pl.ds(t * 16, 16)] = ex
                plsc.store_scatter(exrow, [eids, hq], ex)
                plsc.store_scatter(exrow, [eids, hq + 8], ex * av)
            return _
        lax.fori_loop(0, C // 16, _grp, None)
        pltpu.sync_copy(exrow, acc_sh.at[dstv], add=True)
        pltpu.sync_copy(exhm, ex_out.at[wid, g])
        return _
    lax.fori_loop(0, NCHUNK, _chunk, None)

    plsc.subcore_barrier()

    def _co(t, _):
        j = sid + NS * t
        pltpu.sync_copy(acc_sh.at[pl.ds(j * ZROWS, ZROWS)],
                        acc_out.at[cid, pl.ds(j * ZROWS, ZROWS)])
        return _
    lax.fori_loop(0, nzt, _co, None)


@functools.cache
def _get_passA():
    return functools.partial(
        pl.kernel,
        mesh=plsc.VectorSubcoreMesh(core_axis_name="c", subcore_axis_name="s",
                                    num_cores=NC, num_subcores=NS),
        compiler_params=pltpu.CompilerParams(needs_layout_passes=False,
                                             use_tc_tiling_on_sc=False),
        out_type=[jax.ShapeDtypeStruct((NC, N, 16), _f32),
                  jax.ShapeDtypeStruct((NW, NCHUNK, HEADS, C), _f32)],
        scratch_types=[
            pltpu.VMEM((C,), _i32),
            pltpu.VMEM((C,), _i32),
            pltpu.VMEM((C,), _f32),
            pltpu.VMEM((C, H), _f32),
            pltpu.VMEM((C, H), _f32),
            pltpu.VMEM((C, 16), _f32),
            pltpu.VMEM((C, 16), _f32),
            pltpu.VMEM((HEADS, C), _f32),
            pltpu.VMEM((ZROWS, 16), _f32),
            pltpu.VMEM_SHARED((N, 16), _f32),
            pltpu.SemaphoreType.DMA,
            pltpu.SemaphoreType.DMA,
            pltpu.SemaphoreType.DMA,
        ],
    )(_passA_body)


def _passA(*args):
    return _get_passA()(*args)


def _passB_body(v_hbm, inv_hbm, ex_hbm, src_hbm, dst_hbm,
                out_hbm,
                srcv, dstv, vrows, invrows, exhm, msg, zbuf,
                acc_sh, sem1, sem2):
    cid = lax.axis_index("c")
    sid = lax.axis_index("s")
    wid = sid * NC + cid

    def _zr(i, _):
        for c8 in range(H // 16):
            zbuf[i, pl.ds(c8 * 16, 16)] = jnp.zeros((16,), _f32)
        return _
    lax.fori_loop(0, ZROWS, _zr, None)
    nzt = (NZCH - sid + NS - 1) // NS

    def _zc(t, _):
        j = sid + NS * t
        pltpu.sync_copy(zbuf, acc_sh.at[pl.ds(j * ZROWS, ZROWS)])
        return _
    lax.fori_loop(0, nzt, _zc, None)
    plsc.subcore_barrier()

    def _chunk(g, _):
        base = wid * EPW + g * C
        pltpu.sync_copy(src_hbm.at[pl.ds(base, C)], srcv)
        pltpu.sync_copy(dst_hbm.at[pl.ds(base, C)], dstv)
        pltpu.sync_copy(ex_hbm.at[wid, g], exhm)
        cp1 = pltpu.async_copy(v_hbm.at[srcv], vrows, sem1)
        cp2 = pltpu.async_copy(inv_hbm.at[dstv], invrows, sem2)
        cp1.wait()
        cp2.wait()

        def _grp(t, _):
            eids = lax.iota(_i32, 16) + t * 16
            for h in range(HEADS):
                hq = jnp.full((16,), h, _i32)
                exv = exhm[h, pl.ds(t * 16, 16)]
                dinv = plsc.load_gather(invrows, [eids, hq])
                aatt = exv * dinv
                for d in range(DH):
                    cc = jnp.full((16,), h * DH + d, _i32)
                    vc = plsc.load_gather(vrows, [eids, cc])
                    plsc.store_scatter(msg, [eids, cc], aatt * vc)
            return _
        lax.fori_loop(0, C // 16, _grp, None)
        pltpu.sync_copy(msg, acc_sh.at[dstv], add=True)
        return _
    lax.fori_loop(0, NCHUNK, _chunk, None)

    plsc.subcore_barrier()

    def _co(t, _):
        j = sid + NS * t
        pltpu.sync_copy(acc_sh.at[pl.ds(j * ZROWS, ZROWS)],
                        out_hbm.at[cid, pl.ds(j * ZROWS, ZROWS)])
        return _
    lax.fori_loop(0, nzt, _co, None)


@functools.cache
def _get_passB():
    return functools.partial(
        pl.kernel,
        mesh=plsc.VectorSubcoreMesh(core_axis_name="c", subcore_axis_name="s",
                                    num_cores=NC, num_subcores=NS),
        compiler_params=pltpu.CompilerParams(needs_layout_passes=False,
                                             use_tc_tiling_on_sc=False),
        out_type=jax.ShapeDtypeStruct((NC, N, H), _f32),
        scratch_types=[
            pltpu.VMEM((C,), _i32),
            pltpu.VMEM((C,), _i32),
            pltpu.VMEM((C, H), _f32),
            pltpu.VMEM((C, 16), _f32),
            pltpu.VMEM((HEADS, C), _f32),
            pltpu.VMEM((C, H), _f32),
            pltpu.VMEM((ZROWS, H), _f32),
            pltpu.VMEM_SHARED((N, H), _f32),
            pltpu.SemaphoreType.DMA,
            pltpu.SemaphoreType.DMA,
        ],
    )(_passB_body)


def _passB(*args):
    return _get_passB()(*args)


def _mm(x, W, b, br=1000):
    n, din = x.shape
    dout = W.shape[1]

    def body(x_ref, w_ref, b_ref, o_ref):
        o_ref[...] = jnp.dot(x_ref[...], w_ref[...],
                             preferred_element_type=_f32) + b_ref[...]

    return pl.pallas_call(
        body,
        grid=(n // br,),
        in_specs=[pl.BlockSpec((br, din), lambda i: (i, 0)),
                  pl.BlockSpec((din, dout), lambda i: (0, 0)),
                  pl.BlockSpec((1, dout), lambda i: (0, 0))],
        out_specs=pl.BlockSpec((br, dout), lambda i: (i, 0)),
        out_shape=jax.ShapeDtypeStruct((n, dout), _f32),
    )(x, W, b.reshape(1, -1))


def _invden(acc2, br=2000):
    def body(a_ref, o_ref):
        den = a_ref[0] + a_ref[1]
        inv = 1.0 / (den + 1e-16)
        col = lax.broadcasted_iota(_i32, den.shape, 1)
        o_ref[...] = jnp.where(col < 8, inv, 0.0)

    return pl.pallas_call(
        body,
        grid=(N // br,),
        in_specs=[pl.BlockSpec((NC, br, 16), lambda i: (0, i, 0))],
        out_specs=pl.BlockSpec((br, 16), lambda i: (i, 0)),
        out_shape=jax.ShapeDtypeStruct((N, 16), _f32),
    )(acc2)


def _post(out2, acc2, xr, xres, Ew, Eb, wbO, wbX, ln_g, ln_b,
          W1t, b1, W2t, b2, br=1000):
    def body(o2_ref, a2_ref, xr_ref, xres_ref, ew_ref, eb_ref, wbo_ref,
             wbx_ref, g_ref, be_ref, w1_ref, b1_ref, w2_ref, b2_ref, o_ref):
        a = a2_ref[0] + a2_ref[1]
        den = a[:, 0:8]
        num2 = a[:, 8:16]
        inv = 1.0 / (den + 1e-16)
        s1 = den * inv
        s2 = num2 * inv
        out = (o2_ref[0] + o2_ref[1]
               + jnp.dot(s2, ew_ref[...], preferred_element_type=_f32)
               + jnp.dot(s1, eb_ref[...], preferred_element_type=_f32))
        xr = xr_ref[...]
        logit = (jnp.sum(out * wbo_ref[...], axis=-1, keepdims=True)
                 + jnp.sum(xr * wbx_ref[...], axis=-1, keepdims=True))
        beta = jax.nn.sigmoid(logit)
        y = beta * xr + (1.0 - beta) * out + xres_ref[...]
        m = jnp.mean(y, axis=-1, keepdims=True)
        v = jnp.mean((y - m) ** 2, axis=-1, keepdims=True)
        yn = (y - m) / jnp.sqrt(v + 1e-5) * g_ref[...] + be_ref[...]
        h1 = jnp.dot(yn, w1_ref[...], preferred_element_type=_f32) + b1_ref[...]
        h1 = 0.5 * h1 * (1.0 + lax.erf(h1 * (2.0 ** -0.5)))
        o_ref[...] = (jnp.dot(h1, w2_ref[...], preferred_element_type=_f32)
                      + b2_ref[...] + yn)

    z = lambda i: (0, 0)
    return pl.pallas_call(
        body,
        grid=(N // br,),
        in_specs=[pl.BlockSpec((NC, br, H), lambda i: (0, i, 0)),
                  pl.BlockSpec((NC, br, 16), lambda i: (0, i, 0)),
                  pl.BlockSpec((br, H), lambda i: (i, 0)),
                  pl.BlockSpec((br, H), lambda i: (i, 0)),
                  pl.BlockSpec((HEADS, H), z),
                  pl.BlockSpec((HEADS, H), z),
                  pl.BlockSpec((1, H), z),
                  pl.BlockSpec((1, H), z),
                  pl.BlockSpec((1, H), z),
                  pl.BlockSpec((1, H), z),
                  pl.BlockSpec((H, 4 * H), z),
                  pl.BlockSpec((1, 4 * H), z),
                  pl.BlockSpec((4 * H, H), z),
                  pl.BlockSpec((1, H), z)],
        out_specs=pl.BlockSpec((br, H), lambda i: (i, 0)),
        out_shape=jax.ShapeDtypeStruct((N, H), _f32),
    )(out2, acc2, xr, xres, Ew, Eb, wbO, wbX, ln_g, ln_b,
      W1t, b1.reshape(1, -1), W2t, b2.reshape(1, -1))


def _conv_prep(p):
    w = p['We'][:, 0]
    b = p['be']
    S = jnp.zeros((H, HEADS), _f32).at[jnp.arange(H), jnp.arange(H) // DH].set(1.0)
    Wqw = p['Wq'].T @ (w[:, None] * S)
    Wqb = p['Wq'].T @ (b[:, None] * S)
    bqw = (p['bq'] * w) @ S
    bqb = (p['bq'] * b) @ S
    Ew = (S * w[:, None]).T
    Eb = (S * b[:, None]).T
    wb = p['Wbeta'][0]
    wbO = (wb[:H] + wb[2 * H:]).reshape(1, H)
    wbX = (wb[H:2 * H] - wb[2 * H:]).reshape(1, H)
    return Wqw, Wqb, bqw, bqb, Ew, Eb, wbO, wbX


def kernel(x_user, x_item, edge_index_ui, edge_index_iu,
           edge_attr_ui, edge_attr_iu, params):
    p = params
    src_ui = edge_index_ui[0]
    dst_ui = edge_index_ui[1]
    src_iu = edge_index_iu[0]
    dst_iu = edge_index_iu[1]
    ea_ui = edge_attr_ui[:, 0]
    ea_iu = edge_attr_iu[:, 0]

    xu = _mm(x_user, p['in_user_W'].T, p['in_user_b'])
    xi = _mm(x_item, p['in_item_W'].T, p['in_item_b'])

    for i in range(2):
        pui = p['l%d_ui' % i]
        piu = p['l%d_iu' % i]
        (Wqw_ui, Wqb_ui, bqw_ui, bqb_ui, Ew_ui, Eb_ui, wbO_ui, wbX_ui) = _conv_prep(pui)
        (Wqw_iu, Wqb_iu, bqw_iu, bqb_iu, Ew_iu, Eb_iu, wbO_iu, wbX_iu) = _conv_prep(piu)

        Wcat_u = jnp.concatenate(
            [pui['Wk'].T, pui['Wv'].T, piu['Wq'].T, piu['Wskip'].T,
             Wqw_iu, Wqb_iu], axis=1)
        bcat_u = jnp.concatenate(
            [pui['bk'], pui['bv'], piu['bq'], piu['bskip'], bqw_iu, bqb_iu])
        Wcat_i = jnp.concatenate(
            [piu['Wk'].T, piu['Wv'].T, pui['Wq'].T, pui['Wskip'].T,
             Wqw_ui, Wqb_ui], axis=1)
        bcat_i = jnp.concatenate(
            [piu['bk'], piu['bv'], pui['bq'], pui['bskip'], bqw_ui, bqb_ui])

        yu = _mm(xu, Wcat_u, bcat_u)
        yi = _mm(xi, Wcat_i, bcat_i)
        k_ui, v_ui = yu[:, 0:H], yu[:, H:2 * H]
        q_iu, xr_iu, qwb_iu = yu[:, 2 * H:3 * H], yu[:, 3 * H:4 * H], yu[:, 4 * H:]
        k_iu, v_iu = yi[:, 0:H], yi[:, H:2 * H]
        q_ui, xr_ui, qwb_ui = yi[:, 2 * H:3 * H], yi[:, 3 * H:4 * H], yi[:, 4 * H:]

        acc2_ui, ex_ui = _passA(q_ui, k_ui, qwb_ui, src_ui, dst_ui, ea_ui)
        acc2_iu, ex_iu = _passA(q_iu, k_iu, qwb_iu, src_iu, dst_iu, ea_iu)
        inv_ui = _invden(acc2_ui)
        inv_iu = _invden(acc2_iu)
        out2_ui = _passB(v_ui, inv_ui, ex_ui, src_ui, dst_ui)
        out2_iu = _passB(v_iu, inv_iu, ex_iu, src_iu, dst_iu)

        xi_new = _post(out2_ui, acc2_ui, xr_ui, xi, Ew_ui, Eb_ui, wbO_ui,
                       wbX_ui,
                       p['l%d_ln_item_g' % i].reshape(1, H),
                       p['l%d_ln_item_b' % i].reshape(1, H),
                       p['l%d_ffn_item' % i]['W1'].T,
                       p['l%d_ffn_item' % i]['b1'],
                       p['l%d_ffn_item' % i]['W2'].T,
                       p['l%d_ffn_item' % i]['b2'])
        xu_new = _post(out2_iu, acc2_iu, xr_iu, xu, Ew_iu, Eb_iu, wbO_iu,
                       wbX_iu,
                       p['l%d_ln_user_g' % i].reshape(1, H),
                       p['l%d_ln_user_b' % i].reshape(1, H),
                       p['l%d_ffn_user' % i]['W1'].T,
                       p['l%d_ffn_user' % i]['b1'],
                       p['l%d_ffn_user' % i]['W2'].T,
                       p['l%d_ffn_user' % i]['b2'])
        xu, xi = xu_new, xi_new

    out_u = _mm(xu, p['out_user_W'].T, p['out_user_b'])
    out_i = _mm(xi, p['out_item_W'].T, p['out_item_b'])
    return (out_u, out_i)


# speedup vs baseline: 14.3078x; 1.1919x over previous
import functools

import jax
import jax.numpy as jnp
from jax import lax
from jax.experimental import pallas as pl
from jax.experimental.pallas import tpu as pltpu
from jax.experimental.pallas import tpu_sc as plsc

H = 128
HEADS = 8
DH = 16
N = 10000
E = 320000

NC = 2
NS = 16
NW = NC * NS
CG = 80
NCHG = E // CG
ZROWS = CG
NZCH = N // ZROWS

_f32 = jnp.float32
_i32 = jnp.int32


def _zero_acc(acc_sh, zbuf, sid, width16):
    def _z(i, _):
        for c8 in range(width16):
            zbuf[i, pl.ds(c8 * 16, 16)] = jnp.zeros((16,), _f32)
        return _
    lax.fori_loop(0, ZROWS, _z, None)
    nzt = (NZCH - sid + NS - 1) // NS

    def _zc(t, _):
        j = sid + NS * t
        pltpu.sync_copy(zbuf, acc_sh.at[pl.ds(j * ZROWS, ZROWS)])
        return _
    lax.fori_loop(0, nzt, _zc, None)
    plsc.subcore_barrier()
    return nzt


def _copy_out(acc_sh, out_hbm, cid, sid, nzt):
    def _co(t, _):
        j = sid + NS * t
        pltpu.sync_copy(acc_sh.at[pl.ds(j * ZROWS, ZROWS)],
                        out_hbm.at[cid, pl.ds(j * ZROWS, ZROWS)])
        return _
    lax.fori_loop(0, nzt, _co, None)


def _passA_body(q_hbm, k_hbm, qwb_hbm, ed_hbm,
                acc_out, ex_out,
                ib0, ib1, qr0, qr1, kr0, kr1, wb0, wb1, er0, er1, eh0, eh1,
                ds0, ds1, acc_sh,
                sI0, sI1, sQ0, sQ1, sK0, sK1, sW0, sW1, sS0, sS1, sE0, sE1):
    ib = (ib0, ib1); qr = (qr0, qr1); kr = (kr0, kr1); wb = (wb0, wb1)
    er = (er0, er1); eh = (eh0, eh1); dsb = (ds0, ds1)
    sI = (sI0, sI1); sQ = (sQ0, sQ1); sK = (sK0, sK1); sW = (sW0, sW1)
    sS = (sS0, sS1); sE = (sE0, sE1)
    cid = lax.axis_index("c")
    sid = lax.axis_index("s")
    wid = sid * NC + cid

    nzt = _zero_acc(acc_sh, er0, sid, 1)

    lo = (wid * NCHG) // NW
    cnt = ((wid + 1) * NCHG) // NW - lo

    def idx_issue(j, b):
        pltpu.async_copy(ed_hbm.at[:, pl.ds((lo + j) * CG, CG)], ib[b], sI[b])

    def idx_wait(b):
        pltpu.make_async_copy(ed_hbm.at[:, pl.ds(0, CG)], ib[b], sI[b]).wait()

    def gathers_issue(b):
        pltpu.async_copy(q_hbm.at[ib[b].at[1]], qr[b], sQ[b])
        pltpu.async_copy(k_hbm.at[ib[b].at[0]], kr[b], sK[b])
        pltpu.async_copy(qwb_hbm.at[ib[b].at[1]], wb[b], sW[b])

    def gathers_wait(b):
        pltpu.make_async_copy(q_hbm.at[ib[b].at[1]], qr[b], sQ[b]).wait()
        pltpu.make_async_copy(k_hbm.at[ib[b].at[0]], kr[b], sK[b]).wait()
        pltpu.make_async_copy(qwb_hbm.at[ib[b].at[1]], wb[b], sW[b]).wait()

    def outs_wait(b):
        pltpu.make_async_copy(er[b], acc_sh.at[dsb[b]], sS[b]).wait()
        pltpu.make_async_copy(eh[b], ex_out.at[0], sE[b]).wait()

    def compute(b):
        for t8 in range(CG // 16):
            dsb[b][pl.ds(t8 * 16, 16)] = ib[b][1, pl.ds(t8 * 16, 16)]

        def _grp(t, _):
            eids = lax.iota(_i32, 16) + t * 16
            av = plsc.bitcast(ib[b][2, pl.ds(t * 16, 16)], _f32)
            for h in range(HEADS):
                hq = jnp.full((16,), h, _i32)
                qwv = plsc.load_gather(wb[b], [eids, hq])
                qbv = plsc.load_gather(wb[b], [eids, hq + 8])
                s = qbv + av * qwv
                for d in range(DH):
                    cc = jnp.full((16,), h * DH + d, _i32)
                    qc = plsc.load_gather(qr[b], [eids, cc])
                    kc = plsc.load_gather(kr[b], [eids, cc])
                    s = s + qc * kc
                ex = jnp.exp(s * 0.25)
                eh[b][h, pl.ds(t * 16, 16)] = ex
                plsc.store_scatter(er[b], [eids, hq], ex)
                plsc.store_scatter(er[b], [eids, hq + 8], ex * av)
            return _
        lax.fori_loop(0, CG // 16, _grp, None)

    idx_issue(0, 0)
    idx_wait(0)
    gathers_issue(0)
    idx_issue(1, 1)

    def _iter(j2, _):
        for b in (0, 1):
            j = j2 * 2 + b

            @pl.when(j < cnt)
            def _():
                @pl.when(j + 1 < cnt)
                def _():
                    idx_wait(1 - b)
                    gathers_issue(1 - b)
                gathers_wait(b)

                @pl.when(j2 >= 1)
                def _():
                    outs_wait(b)
                compute(b)
                pltpu.async_copy(er[b], acc_sh.at[dsb[b]], sS[b], add=True)
                pltpu.async_copy(eh[b], ex_out.at[lo + j], sE[b])

                @pl.when(j + 2 < cnt)
                def _():
                    idx_issue(j + 2, b)
        return _
    lax.fori_loop(0, (cnt + 1) // 2, _iter, None)
    outs_wait(0)
    outs_wait(1)

    plsc.subcore_barrier()
    _copy_out(acc_sh, acc_out, cid, sid, nzt)


@functools.cache
def _get_passA():
    return functools.partial(
        pl.kernel,
        mesh=plsc.VectorSubcoreMesh(core_axis_name="c", subcore_axis_name="s",
                                    num_cores=NC, num_subcores=NS),
        compiler_params=pltpu.CompilerParams(needs_layout_passes=False,
                                             use_tc_tiling_on_sc=False),
        out_type=[jax.ShapeDtypeStruct((NC, N, 16), _f32),
                  jax.ShapeDtypeStruct((NCHG, HEADS, CG), _f32)],
        scratch_types=[
            pltpu.VMEM((3, CG), _i32),
            pltpu.VMEM((3, CG), _i32),
            pltpu.VMEM((CG, H), _f32),
            pltpu.VMEM((CG, H), _f32),
            pltpu.VMEM((CG, H), _f32),
            pltpu.VMEM((CG, H), _f32),
            pltpu.VMEM((CG, 16), _f32),
            pltpu.VMEM((CG, 16), _f32),
            pltpu.VMEM((CG, 16), _f32),
            pltpu.VMEM((CG, 16), _f32),
            pltpu.VMEM((HEADS, CG), _f32),
            pltpu.VMEM((HEADS, CG), _f32),
            pltpu.VMEM((CG,), _i32),
            pltpu.VMEM((CG,), _i32),
            pltpu.VMEM_SHARED((N, 16), _f32),
        ] + [pltpu.SemaphoreType.DMA] * 12,
    )(_passA_body)


def _passA(*args):
    return _get_passA()(*args)


def _passB_body(v_hbm, inv_hbm, ex_hbm, ed_hbm,
                out_hbm,
                ib0, ib1, vr0, vr1, ir0, ir1, xm0, xm1, mg0, mg1, ds0, ds1,
                acc_sh,
                sI0, sI1, sV0, sV1, sR0, sR1, sX0, sX1, sS0, sS1):
    ib = (ib0, ib1); vr = (vr0, vr1); ir = (ir0, ir1); xm = (xm0, xm1)
    mg = (mg0, mg1); dsb = (ds0, ds1)
    sI = (sI0, sI1); sV = (sV0, sV1); sR = (sR0, sR1); sX = (sX0, sX1)
    sS = (sS0, sS1)
    cid = lax.axis_index("c")
    sid = lax.axis_index("s")
    wid = sid * NC + cid

    nzt = _zero_acc(acc_sh, mg0, sid, H // 16)

    lo = (wid * NCHG) // NW
    cnt = ((wid + 1) * NCHG) // NW - lo

    def idx_issue(j, b):
        pltpu.async_copy(ed_hbm.at[:, pl.ds((lo + j) * CG, CG)], ib[b], sI[b])
        pltpu.async_copy(ex_hbm.at[lo + j], xm[b], sX[b])

    def idx_wait(b):
        pltpu.make_async_copy(ed_hbm.at[:, pl.ds(0, CG)], ib[b], sI[b]).wait()

    def gathers_issue(b):
        pltpu.async_copy(v_hbm.at[ib[b].at[0]], vr[b], sV[b])
        pltpu.async_copy(inv_hbm.at[ib[b].at[1]], ir[b], sR[b])

    def gathers_wait(b):
        pltpu.make_async_copy(v_hbm.at[ib[b].at[0]], vr[b], sV[b]).wait()
        pltpu.make_async_copy(inv_hbm.at[ib[b].at[1]], ir[b], sR[b]).wait()
        pltpu.make_async_copy(ex_hbm.at[0], xm[b], sX[b]).wait()

    def outs_wait(b):
        pltpu.make_async_copy(mg[b], acc_sh.at[dsb[b]], sS[b]).wait()

    def compute(b):
        for t8 in range(CG // 16):
            dsb[b][pl.ds(t8 * 16, 16)] = ib[b][1, pl.ds(t8 * 16, 16)]

        def _grp(t, _):
            eids = lax.iota(_i32, 16) + t * 16
            for h in range(HEADS):
                hq = jnp.full((16,), h, _i32)
                exv = xm[b][h, pl.ds(t * 16, 16)]
                dinv = plsc.load_gather(ir[b], [eids, hq])
                aatt = exv * dinv
                for d in range(DH):
                    cc = jnp.full((16,), h * DH + d, _i32)
                    vc = plsc.load_gather(vr[b], [eids, cc])
                    plsc.store_scatter(mg[b], [eids, cc], aatt * vc)
            return _
        lax.fori_loop(0, CG // 16, _grp, None)

    idx_issue(0, 0)
    idx_wait(0)
    gathers_issue(0)
    idx_issue(1, 1)

    def _iter(j2, _):
        for b in (0, 1):
            j = j2 * 2 + b

            @pl.when(j < cnt)
            def _():
                @pl.when(j + 1 < cnt)
                def _():
                    idx_wait(1 - b)
                    gathers_issue(1 - b)
                gathers_wait(b)

                @pl.when(j2 >= 1)
                def _():
                    outs_wait(b)
                compute(b)
                pltpu.async_copy(mg[b], acc_sh.at[dsb[b]], sS[b], add=True)

                @pl.when(j + 2 < cnt)
                def _():
                    idx_issue(j + 2, b)
        return _
    lax.fori_loop(0, (cnt + 1) // 2, _iter, None)
    outs_wait(0)
    outs_wait(1)

    plsc.subcore_barrier()
    _copy_out(acc_sh, out_hbm, cid, sid, nzt)


@functools.cache
def _get_passB():
    return functools.partial(
        pl.kernel,
        mesh=plsc.VectorSubcoreMesh(core_axis_name="c", subcore_axis_name="s",
                                    num_cores=NC, num_subcores=NS),
        compiler_params=pltpu.CompilerParams(needs_layout_passes=False,
                                             use_tc_tiling_on_sc=False),
        out_type=jax.ShapeDtypeStruct((NC, N, H), _f32),
        scratch_types=[
            pltpu.VMEM((3, CG), _i32),
            pltpu.VMEM((3, CG), _i32),
            pltpu.VMEM((CG, H), _f32),
            pltpu.VMEM((CG, H), _f32),
            pltpu.VMEM((CG, 16), _f32),
            pltpu.VMEM((CG, 16), _f32),
            pltpu.VMEM((HEADS, CG), _f32),
            pltpu.VMEM((HEADS, CG), _f32),
            pltpu.VMEM((CG, H), _f32),
            pltpu.VMEM((CG, H), _f32),
            pltpu.VMEM((CG,), _i32),
            pltpu.VMEM((CG,), _i32),
            pltpu.VMEM_SHARED((N, H), _f32),
        ] + [pltpu.SemaphoreType.DMA] * 10,
    )(_passB_body)


def _passB(*args):
    return _get_passB()(*args)


def _mm(x, W, b, br=1000):
    n, din = x.shape
    dout = W.shape[1]

    def body(x_ref, w_ref, b_ref, o_ref):
        o_ref[...] = jnp.dot(x_ref[...], w_ref[...],
                             preferred_element_type=_f32) + b_ref[...]

    return pl.pallas_call(
        body,
        grid=(n // br,),
        in_specs=[pl.BlockSpec((br, din), lambda i: (i, 0)),
                  pl.BlockSpec((din, dout), lambda i: (0, 0)),
                  pl.BlockSpec((1, dout), lambda i: (0, 0))],
        out_specs=pl.BlockSpec((br, dout), lambda i: (i, 0)),
        out_shape=jax.ShapeDtypeStruct((n, dout), _f32),
    )(x, W, b.reshape(1, -1))


def _invden(acc2, br=2000):
    def body(a_ref, o_ref):
        den = a_ref[0] + a_ref[1]
        inv = 1.0 / (den + 1e-16)
        col = lax.broadcasted_iota(_i32, den.shape, 1)
        o_ref[...] = jnp.where(col < 8, inv, 0.0)

    return pl.pallas_call(
        body,
        grid=(N // br,),
        in_specs=[pl.BlockSpec((NC, br, 16), lambda i: (0, i, 0))],
        out_specs=pl.BlockSpec((br, 16), lambda i: (i, 0)),
        out_shape=jax.ShapeDtypeStruct((N, 16), _f32),
    )(acc2)


def _post(out2, acc2, xr, xres, Ew, Eb, wbO, wbX, ln_g, ln_b,
          W1t, b1, W2t, b2, br=1000):
    def body(o2_ref, a2_ref, xr_ref, xres_ref, ew_ref, eb_ref, wbo_ref,
             wbx_ref, g_ref, be_ref, w1_ref, b1_ref, w2_ref, b2_ref, o_ref):
        a = a2_ref[0] + a2_ref[1]
        den = a[:, 0:8]
        num2 = a[:, 8:16]
        inv = 1.0 / (den + 1e-16)
        s1 = den * inv
        s2 = num2 * inv
        out = (o2_ref[0] + o2_ref[1]
               + jnp.dot(s2, ew_ref[...], preferred_element_type=_f32)
               + jnp.dot(s1, eb_ref[...], preferred_element_type=_f32))
        xr = xr_ref[...]
        logit = (jnp.sum(out * wbo_ref[...], axis=-1, keepdims=True)
                 + jnp.sum(xr * wbx_ref[...], axis=-1, keepdims=True))
        beta = jax.nn.sigmoid(logit)
        y = beta * xr + (1.0 - beta) * out + xres_ref[...]
        m = jnp.mean(y, axis=-1, keepdims=True)
        v = jnp.mean((y - m) ** 2, axis=-1, keepdims=True)
        yn = (y - m) / jnp.sqrt(v + 1e-5) * g_ref[...] + be_ref[...]
        h1 = jnp.dot(yn, w1_ref[...], preferred_element_type=_f32) + b1_ref[...]
        h1 = 0.5 * h1 * (1.0 + lax.erf(h1 * (2.0 ** -0.5)))
        o_ref[...] = (jnp.dot(h1, w2_ref[...], preferred_element_type=_f32)
                      + b2_ref[...] + yn)

    z = lambda i: (0, 0)
    return pl.pallas_call(
        body,
        grid=(N // br,),
        in_specs=[pl.BlockSpec((NC, br, H), lambda i: (0, i, 0)),
                  pl.BlockSpec((NC, br, 16), lambda i: (0, i, 0)),
                  pl.BlockSpec((br, H), lambda i: (i, 0)),
                  pl.BlockSpec((br, H), lambda i: (i, 0)),
                  pl.BlockSpec((HEADS, H), z),
                  pl.BlockSpec((HEADS, H), z),
                  pl.BlockSpec((1, H), z),
                  pl.BlockSpec((1, H), z),
                  pl.BlockSpec((1, H), z),
                  pl.BlockSpec((1, H), z),
                  pl.BlockSpec((H, 4 * H), z),
                  pl.BlockSpec((1, 4 * H), z),
                  pl.BlockSpec((4 * H, H), z),
                  pl.BlockSpec((1, H), z)],
        out_specs=pl.BlockSpec((br, H), lambda i: (i, 0)),
        out_shape=jax.ShapeDtypeStruct((N, H), _f32),
    )(out2, acc2, xr, xres, Ew, Eb, wbO, wbX, ln_g, ln_b,
      W1t, b1.reshape(1, -1), W2t, b2.reshape(1, -1))


def _conv_prep(p):
    w = p['We'][:, 0]
    b = p['be']
    S = jnp.zeros((H, HEADS), _f32).at[jnp.arange(H), jnp.arange(H) // DH].set(1.0)
    Wqw = p['Wq'].T @ (w[:, None] * S)
    Wqb = p['Wq'].T @ (b[:, None] * S)
    bqw = (p['bq'] * w) @ S
    bqb = (p['bq'] * b) @ S
    Ew = (S * w[:, None]).T
    Eb = (S * b[:, None]).T
    wb = p['Wbeta'][0]
    wbO = (wb[:H] + wb[2 * H:]).reshape(1, H)
    wbX = (wb[H:2 * H] - wb[2 * H:]).reshape(1, H)
    return Wqw, Wqb, bqw, bqb, Ew, Eb, wbO, wbX


def kernel(x_user, x_item, edge_index_ui, edge_index_iu,
           edge_attr_ui, edge_attr_iu, params):
    p = params
    ed_ui = jnp.concatenate(
        [edge_index_ui.astype(_i32),
         lax.bitcast_convert_type(edge_attr_ui[:, 0], _i32)[None]], axis=0)
    ed_iu = jnp.concatenate(
        [edge_index_iu.astype(_i32),
         lax.bitcast_convert_type(edge_attr_iu[:, 0], _i32)[None]], axis=0)

    xu = _mm(x_user, p['in_user_W'].T, p['in_user_b'])
    xi = _mm(x_item, p['in_item_W'].T, p['in_item_b'])

    for i in range(2):
        pui = p['l%d_ui' % i]
        piu = p['l%d_iu' % i]
        (Wqw_ui, Wqb_ui, bqw_ui, bqb_ui, Ew_ui, Eb_ui, wbO_ui, wbX_ui) = _conv_prep(pui)
        (Wqw_iu, Wqb_iu, bqw_iu, bqb_iu, Ew_iu, Eb_iu, wbO_iu, wbX_iu) = _conv_prep(piu)

        Wcat_u = jnp.concatenate(
            [pui['Wk'].T, pui['Wv'].T, piu['Wq'].T, piu['Wskip'].T,
             Wqw_iu, Wqb_iu], axis=1)
        bcat_u = jnp.concatenate(
            [pui['bk'], pui['bv'], piu['bq'], piu['bskip'], bqw_iu, bqb_iu])
        Wcat_i = jnp.concatenate(
            [piu['Wk'].T, piu['Wv'].T, pui['Wq'].T, pui['Wskip'].T,
             Wqw_ui, Wqb_ui], axis=1)
        bcat_i = jnp.concatenate(
            [piu['bk'], piu['bv'], pui['bq'], pui['bskip'], bqw_ui, bqb_ui])

        yu = _mm(xu, Wcat_u, bcat_u)
        yi = _mm(xi, Wcat_i, bcat_i)
        k_ui, v_ui = yu[:, 0:H], yu[:, H:2 * H]
        q_iu, xr_iu, qwb_iu = yu[:, 2 * H:3 * H], yu[:, 3 * H:4 * H], yu[:, 4 * H:]
        k_iu, v_iu = yi[:, 0:H], yi[:, H:2 * H]
        q_ui, xr_ui, qwb_ui = yi[:, 2 * H:3 * H], yi[:, 3 * H:4 * H], yi[:, 4 * H:]

        acc2_ui, ex_ui = _passA(q_ui, k_ui, qwb_ui, ed_ui)
        acc2_iu, ex_iu = _passA(q_iu, k_iu, qwb_iu, ed_iu)
        inv_ui = _invden(acc2_ui)
        inv_iu = _invden(acc2_iu)
        out2_ui = _passB(v_ui, inv_ui, ex_ui, ed_ui)
        out2_iu = _passB(v_iu, inv_iu, ex_iu, ed_iu)

        xi_new = _post(out2_ui, acc2_ui, xr_ui, xi, Ew_ui, Eb_ui, wbO_ui,
                       wbX_ui,
                       p['l%d_ln_item_g' % i].reshape(1, H),
                       p['l%d_ln_item_b' % i].reshape(1, H),
                       p['l%d_ffn_item' % i]['W1'].T,
                       p['l%d_ffn_item' % i]['b1'],
                       p['l%d_ffn_item' % i]['W2'].T,
                       p['l%d_ffn_item' % i]['b2'])
        xu_new = _post(out2_iu, acc2_iu, xr_iu, xu, Ew_iu, Eb_iu, wbO_iu,
                       wbX_iu,
                       p['l%d_ln_user_g' % i].reshape(1, H),
                       p['l%d_ln_user_b' % i].reshape(1, H),
                       p['l%d_ffn_user' % i]['W1'].T,
                       p['l%d_ffn_user' % i]['b1'],
                       p['l%d_ffn_user' % i]['W2'].T,
                       p['l%d_ffn_user' % i]['b2'])
        xu, xi = xu_new, xi_new

    out_u = _mm(xu, p['out_user_W'].T, p['out_user_b'])
    out_i = _mm(xi, p['out_item_W'].T, p['out_item_b'])
    return (out_u, out_i)
